# baseline (speedup 1.0000x reference)
import functools

import jax
import jax.numpy as jnp
from jax import lax
from jax.experimental import pallas as pl
from jax.experimental.pallas import tpu as pltpu
from jax.experimental.pallas import tpu_sc as plsc


@functools.lru_cache(maxsize=None)
def _make_gather(V, D, T, Bt):
    info = plsc.get_sparse_core_info()
    NC, NS, L = info.num_cores, info.num_subcores, info.num_lanes
    NW = NC * NS
    C = 256
    QP = Bt // C
    n_units = T * QP
    assert n_units % NW == 0
    per_w = n_units // NW
    assert per_w % 2 == 0
    mesh = plsc.VectorSubcoreMesh(core_axis_name="c", subcore_axis_name="s")

    @functools.partial(
        pl.kernel,
        mesh=mesh,
        out_type=jax.ShapeDtypeStruct((T, D, Bt), jnp.float32),
        scratch_types=[
            pltpu.VMEM((1, C), jnp.int32),
            pltpu.VMEM((1, C), jnp.int32),
            pltpu.VMEM((C,), jnp.int32),
            pltpu.VMEM((C,), jnp.int32),
            pltpu.VMEM((C,), jnp.int32),
            pltpu.VMEM((C,), jnp.int32),
            pltpu.VMEM((C, 128), jnp.float32),
            pltpu.VMEM((1, D, C), jnp.float32),
            pltpu.VMEM((1, D, C), jnp.float32),
            pltpu.SemaphoreType.DMA,
            pltpu.SemaphoreType.DMA,
            pltpu.SemaphoreType.DMA,
            pltpu.SemaphoreType.DMA,
            pltpu.SemaphoreType.DMA,
        ],
        compiler_params=pltpu.CompilerParams(
            use_tc_tiling_on_sc=True, needs_layout_passes=False
        ),
    )
    def gather(
        t128_hbm, idx_hbm, out_hbm,
        idxv0, idxv1, q0, q1, r0, r1, rows, outv0, outv1,
        isem0, isem1, gsem, wsem0, wsem1,
    ):
        idxv = (idxv0, idxv1)
        qv = (q0, q1)
        rv = (r0, r1)
        outv = (outv0, outv1)
        isem = (isem0, isem1)
        wsem = (wsem0, wsem1)
        wid = lax.axis_index("s") * NC + lax.axis_index("c")
        iota = lax.iota(jnp.int32, L)

        def unit(k):
            u = wid + k * NW
            return u // QP, lax.rem(u, QP) * C

        def idx_slice(k):
            t, c0 = unit(k)
            return idx_hbm.at[pl.ds(t, 1), pl.ds(c0, C)]

        def out_slice(k):
            t, c0 = unit(k)
            return out_hbm.at[pl.ds(t, 1), :, pl.ds(c0, C)]

        def start_idx(k, s):
            pltpu.async_copy(idx_slice(k), idxv[s], isem[s])

        def wait_idx(k, s):
            pltpu.make_async_copy(idx_slice(k), idxv[s], isem[s]).wait()

        def split_idx(s):
            for jb in range(C // L):
                v = idxv[s][0, pl.ds(jb * L, L)]
                qv[s][pl.ds(jb * L, L)] = lax.shift_right_logical(v, 2)
                rv[s][pl.ds(jb * L, L)] = lax.bitwise_and(v, 3)

        def start_gather(s):
            pltpu.async_copy(t128_hbm.at[qv[s]], rows, gsem)

        def wait_gather(s):
            pltpu.make_async_copy(t128_hbm.at[qv[s]], rows, gsem).wait()

        def extract(s):
            for jb in range(C // L):
                rem = rv[s][pl.ds(jb * L, L)]
                ridx = iota + (jb * L)
                cbase = rem * D
                for d in range(D):
                    v = plsc.load_gather(rows, [ridx, cbase + d])
                    outv[s][0, d, pl.ds(jb * L, L)] = v

        def start_wb(k, s):
            pltpu.async_copy(outv[s], out_slice(k), wsem[s])

        def wait_wb(k, s):
            pltpu.make_async_copy(outv[s], out_slice(k), wsem[s]).wait()

        n_it = per_w // 2

        start_idx(0, 0)
        start_idx(1, 1)
        wait_idx(0, 0)
        split_idx(0)
        start_gather(0)
        wait_idx(1, 1)
        split_idx(1)

        def body(i, carry):
            a = 2 * i
            b = a + 1
            wait_gather(0)

            @pl.when(i > 0)
            def _():
                wait_wb(a - 2, 0)

            extract(0)
            start_gather(1)
            start_wb(a, 0)

            @pl.when(i + 1 < n_it)
            def _():
                start_idx(a + 2, 0)

            wait_gather(1)

            @pl.when(i > 0)
            def _():
                wait_wb(b - 2, 1)

            extract(1)

            @pl.when(i + 1 < n_it)
            def _():
                wait_idx(a + 2, 0)

            split_idx(0)

            @pl.when(i + 1 < n_it)
            def _():
                start_gather(0)

            start_wb(b, 1)

            @pl.when(i + 1 < n_it)
            def _():
                start_idx(b + 2, 1)
                wait_idx(b + 2, 1)
                split_idx(1)

            return carry

        lax.fori_loop(0, n_it, body, 0)
        wait_wb(per_w - 2, 0)
        wait_wb(per_w - 1, 1)

    return gather


def kernel(input, table):
    T, Bt = input.shape
    V, D = table.shape
    t128 = table.reshape(V * D // 128, 128)
    out = _make_gather(V, D, T, Bt)(t128, input)
    return jnp.transpose(out, (0, 2, 1))


# speedup vs baseline: 1.1779x; 1.1779x over previous
import functools

import jax
import jax.numpy as jnp
from jax import lax
from jax.experimental import pallas as pl
from jax.experimental.pallas import tpu as pltpu
from jax.experimental.pallas import tpu_sc as plsc


@functools.lru_cache(maxsize=None)
def _make_pretranspose(V, D):
    info = plsc.get_sparse_core_info()
    NC, NS, L = info.num_cores, info.num_subcores, info.num_lanes
    NW = NC * NS
    K = 1280
    NCH = V // K
    TAIL = V - NCH * K
    trips = -(-NCH // NW)
    mesh = plsc.VectorSubcoreMesh(core_axis_name="c", subcore_axis_name="s")

    @functools.partial(
        pl.kernel,
        mesh=mesh,
        out_type=jax.ShapeDtypeStruct((V * D,), jnp.float32),
        scratch_types=[
            pltpu.VMEM((D, K), jnp.float32),
            pltpu.VMEM((K * D,), jnp.float32),
            pltpu.VMEM((max(TAIL, 1) * D,), jnp.float32),
            pltpu.SemaphoreType.DMA,
            pltpu.SemaphoreType.DMA,
        ],
        compiler_params=pltpu.CompilerParams(
            use_tc_tiling_on_sc=True, needs_layout_passes=False
        ),
    )
    def pret(tT_hbm, tail_hbm, out_hbm, inv, outv, inv2, isem, osem):
        wid = lax.axis_index("s") * NC + lax.axis_index("c")
        iv32 = lax.iota(jnp.int32, L) * D

        def in_slice(c):
            return tT_hbm.at[:, pl.ds(c * K, K)]

        def out_slice(c):
            return out_hbm.at[pl.ds(c * (K * D), K * D)]

        def transpose_chunk(ncols):
            def jbody(jb, carry):
                jc = jb * (L * D)
                src = jb * L
                for d in range(D):
                    v = inv[d, pl.ds(src, L)]
                    plsc.store_scatter(outv, [iv32 + (jc + d)], v)
                return carry

            lax.fori_loop(0, ncols // L, jbody, 0)

        def trip(k, first, last):
            c = wid + k * NW

            @pl.when(c < NCH)
            def _():
                pltpu.make_async_copy(in_slice(c), inv, isem).wait()
                if not first:
                    pltpu.make_async_copy(outv, out_slice(c - NW), osem).wait()
                transpose_chunk(K)
                if not last:

                    @pl.when(c + NW < NCH)
                    def _():
                        pltpu.async_copy(in_slice(c + NW), inv, isem)
                pltpu.async_copy(outv, out_slice(c), osem)

        pltpu.async_copy(in_slice(wid), inv, isem)
        for k in range(trips):
            trip(k, k == 0, k == trips - 1)

        clast = wid + (trips - 1) * NW

        @pl.when(clast < NCH)
        def _():
            pltpu.make_async_copy(outv, out_slice(clast), osem).wait()

        @pl.when(clast >= NCH)
        def _():
            pltpu.make_async_copy(
                outv, out_slice(wid + (trips - 2) * NW), osem
            ).wait()

        if TAIL:
            nt = TAIL * D

            @pl.when(wid == 0)
            def _():
                pltpu.async_copy(tail_hbm, inv2, isem)
                pltpu.make_async_copy(tail_hbm, inv2, isem).wait()
                pltpu.async_copy(inv2, out_hbm.at[pl.ds(NCH * K * D, nt)], osem)
                pltpu.make_async_copy(
                    inv2, out_hbm.at[pl.ds(NCH * K * D, nt)], osem
                ).wait()

    return pret


@functools.lru_cache(maxsize=None)
def _make_gather(V, D, T, Bt):
    info = plsc.get_sparse_core_info()
    NC, NS = info.num_cores, info.num_subcores
    NW = NC * NS
    C = 1024
    while Bt % C != 0:
        C //= 2
    QP = Bt // C
    n_units = T * QP
    assert n_units % NW == 0
    per_w = n_units // NW
    mesh = plsc.VectorSubcoreMesh(core_axis_name="c", subcore_axis_name="s")

    @functools.partial(
        pl.kernel,
        mesh=mesh,
        out_type=jax.ShapeDtypeStruct((T, Bt, D), jnp.float32),
        scratch_types=[
            pltpu.VMEM((1, C), jnp.int32),
            pltpu.VMEM((1, C), jnp.int32),
            pltpu.VMEM((1, C, D), jnp.float32),
            pltpu.VMEM((1, C, D), jnp.float32),
            pltpu.SemaphoreType.DMA,
            pltpu.SemaphoreType.DMA,
            pltpu.SemaphoreType.DMA,
            pltpu.SemaphoreType.DMA,
            pltpu.SemaphoreType.DMA,
            pltpu.SemaphoreType.DMA,
        ],
        compiler_params=pltpu.CompilerParams(use_tc_tiling_on_sc=False),
    )
    def gather(
        table_hbm, idx_hbm, out_hbm,
        idx_v0, idx_v1, rows_v0, rows_v1,
        isem0, isem1, gsem0, gsem1, wsem0, wsem1,
    ):
        idx_v = (idx_v0, idx_v1)
        rows_v = (rows_v0, rows_v1)
        isem = (isem0, isem1)
        gsem = (gsem0, gsem1)
        wsem = (wsem0, wsem1)
        wid = lax.axis_index("s") * NC + lax.axis_index("c")

        def unit(k):
            u = wid + k * NW
            return u // QP, lax.rem(u, QP) * C

        def idx_slice(k):
            t, c0 = unit(k)
            return idx_hbm.at[pl.ds(t, 1), pl.ds(c0, C)]

        def out_slice(k):
            t, c0 = unit(k)
            return out_hbm.at[pl.ds(t, 1), pl.ds(c0, C), :]

        def start_idx(k):
            pltpu.async_copy(idx_slice(k), idx_v[k % 2], isem[k % 2])

        def wait_idx(k):
            pltpu.make_async_copy(idx_slice(k), idx_v[k % 2], isem[k % 2]).wait()

        def start_gather(k):
            s = k % 2
            pltpu.async_copy(table_hbm.at[idx_v[s].at[0]], rows_v[s].at[0], gsem[s])

        def wait_gather(k):
            s = k % 2
            pltpu.make_async_copy(
                table_hbm.at[idx_v[s].at[0]], rows_v[s].at[0], gsem[s]
            ).wait()

        def start_wb(k):
            pltpu.async_copy(rows_v[k % 2], out_slice(k), wsem[k % 2])

        def wait_wb(k):
            pltpu.make_async_copy(rows_v[k % 2], out_slice(k), wsem[k % 2]).wait()

        start_idx(0)
        if per_w > 1:
            start_idx(1)
        wait_idx(0)
        start_gather(0)
        for k in range(per_w):
            if k + 1 < per_w:
                wait_idx(k + 1)
                if k + 1 >= 2:
                    wait_wb(k - 1)
                start_gather(k + 1)
            wait_gather(k)
            if k + 2 < per_w:
                start_idx(k + 2)
            start_wb(k)
        for k in (per_w - 2, per_w - 1):
            if k >= 0:
                wait_wb(k)

    return gather


def kernel(input, table):
    T, Bt = input.shape
    V, D = table.shape
    K = 1280
    tail = table[(V // K) * K :, :].reshape(-1)
    flat = _make_pretranspose(V, D)(table.T, tail)
    t2 = flat.reshape(V, D)
    return _make_gather(V, D, T, Bt)(t2, input)


# speedup vs baseline: 1.4653x; 1.2440x over previous
import functools

import jax
import jax.numpy as jnp
from jax import lax
from jax.experimental import pallas as pl
from jax.experimental.pallas import tpu as pltpu
from jax.experimental.pallas import tpu_sc as plsc


@functools.lru_cache(maxsize=None)
def _make_gather(V, D, T, Bt):
    info = plsc.get_sparse_core_info()
    NC, NS = info.num_cores, info.num_subcores
    NW = NC * NS
    C = 1024
    while Bt % C != 0:
        C //= 2
    QP = Bt // C
    n_units = T * QP
    assert n_units % NW == 0
    per_w = n_units // NW
    mesh = plsc.VectorSubcoreMesh(core_axis_name="c", subcore_axis_name="s")

    @functools.partial(
        pl.kernel,
        mesh=mesh,
        out_type=jax.ShapeDtypeStruct((T, Bt, D), jnp.float32),
        scratch_types=[
            pltpu.VMEM((1, C), jnp.int32),
            pltpu.VMEM((1, C), jnp.int32),
            pltpu.VMEM((1, C, D), jnp.float32),
            pltpu.VMEM((1, C, D), jnp.float32),
            pltpu.SemaphoreType.DMA,
            pltpu.SemaphoreType.DMA,
            pltpu.SemaphoreType.DMA,
            pltpu.SemaphoreType.DMA,
            pltpu.SemaphoreType.DMA,
            pltpu.SemaphoreType.DMA,
        ],
        compiler_params=pltpu.CompilerParams(use_tc_tiling_on_sc=False),
    )
    def gather(
        table_hbm, idx_hbm, out_hbm,
        idx_v0, idx_v1, rows_v0, rows_v1,
        isem0, isem1, gsem0, gsem1, wsem0, wsem1,
    ):
        idx_v = (idx_v0, idx_v1)
        rows_v = (rows_v0, rows_v1)
        isem = (isem0, isem1)
        gsem = (gsem0, gsem1)
        wsem = (wsem0, wsem1)
        wid = lax.axis_index("s") * NC + lax.axis_index("c")

        def unit(k):
            u = wid + k * NW
            return u // QP, lax.rem(u, QP) * C

        def idx_slice(k):
            t, c0 = unit(k)
            return idx_hbm.at[pl.ds(t, 1), pl.ds(c0, C)]

        def out_slice(k):
            t, c0 = unit(k)
            return out_hbm.at[pl.ds(t, 1), pl.ds(c0, C), :]

        def start_idx(k):
            pltpu.async_copy(idx_slice(k), idx_v[k % 2], isem[k % 2])

        def wait_idx(k):
            pltpu.make_async_copy(idx_slice(k), idx_v[k % 2], isem[k % 2]).wait()

        def start_gather(k):
            s = k % 2
            pltpu.async_copy(table_hbm.at[idx_v[s].at[0]], rows_v[s].at[0], gsem[s])

        def wait_gather(k):
            s = k % 2
            pltpu.make_async_copy(
                table_hbm.at[idx_v[s].at[0]], rows_v[s].at[0], gsem[s]
            ).wait()

        def start_wb(k):
            pltpu.async_copy(rows_v[k % 2], out_slice(k), wsem[k % 2])

        def wait_wb(k):
            pltpu.make_async_copy(rows_v[k % 2], out_slice(k), wsem[k % 2]).wait()

        start_idx(0)
        if per_w > 1:
            start_idx(1)
        wait_idx(0)
        start_gather(0)
        for k in range(per_w):
            if k + 1 < per_w:
                wait_idx(k + 1)
                if k + 1 >= 2:
                    wait_wb(k - 1)
                start_gather(k + 1)
            wait_gather(k)
            if k + 2 < per_w:
                start_idx(k + 2)
            start_wb(k)
        for k in (per_w - 2, per_w - 1):
            if k >= 0:
                wait_wb(k)

    return gather


def kernel(input, table):
    T, Bt = input.shape
    V, D = table.shape
    return _make_gather(V, D, T, Bt)(table, input)
